# baseline (speedup 1.0000x reference)
import functools

import jax
import jax.numpy as jnp
from jax import lax
from jax.experimental import pallas as pl
from jax.experimental.pallas import tpu as pltpu
from jax.experimental.pallas import tpu_sc as plsc

BLK = 128
NUM_EXPERTS = 16
NEG = -1e30


def _gating_body(xf_ref, wg1_ref, wg2_ref, gates_ref, dest_ref, bte_ref,
                 loss_ref):
    T = xf_ref.shape[0]
    E = NUM_EXPERTS
    A = 2 * T
    nblk = A // BLK
    nb_out = bte_ref.shape[0]

    xf = xf_ref[...]
    h1 = jnp.maximum(jnp.dot(xf, wg1_ref[...]), 0.0)
    logits = jnp.dot(h1, wg2_ref[...])

    m = jnp.max(logits, axis=-1, keepdims=True)
    el = jnp.exp(logits - m)
    probs = el / jnp.sum(el, axis=-1, keepdims=True)

    iota = lax.broadcasted_iota(jnp.int32, (T, E), 1)
    i1 = jnp.min(jnp.where(logits == m, iota, E), axis=-1, keepdims=True)
    masked = jnp.where(iota == i1, NEG, logits)
    m2 = jnp.max(masked, axis=-1, keepdims=True)
    i2 = jnp.min(jnp.where(masked == m2, iota, E), axis=-1, keepdims=True)
    p1 = jnp.sum(jnp.where(iota == i1, probs, 0.0), axis=-1, keepdims=True)
    p2 = jnp.sum(jnp.where(iota == i2, probs, 0.0), axis=-1, keepdims=True)
    den = p1 + p2 + 1e-9
    gates_ref[...] = jnp.concatenate([p1 / den, p2 / den], axis=-1)

    eps = 1e-9
    p_bar = jnp.sum(probs, axis=0, keepdims=True) * (1.0 / T)
    h_marg = -jnp.sum(p_bar * jnp.log(p_bar + eps))
    h_cond = jnp.sum(-(probs * jnp.log(probs + eps))) * (1.0 / T)
    loss_ref[...] = jnp.reshape(-h_marg + h_cond, (1, 1))

    iota_a = lax.broadcasted_iota(jnp.int32, (A, E), 1)
    e_flat = jnp.concatenate([i1, i2], axis=0)
    onehot = (e_flat == iota_a).astype(jnp.float32)

    ri = lax.broadcasted_iota(jnp.int32, (BLK, BLK), 0)
    ci = lax.broadcasted_iota(jnp.int32, (BLK, BLK), 1)
    tril = (ri >= ci).astype(jnp.bfloat16)

    onehot_b = onehot.astype(jnp.bfloat16)
    cum_blocks = []
    run = jnp.zeros((1, E), jnp.float32)
    for j in range(nblk):
        oj = onehot[j * BLK:(j + 1) * BLK]
        cj = jnp.dot(tril, onehot_b[j * BLK:(j + 1) * BLK],
                     preferred_element_type=jnp.float32)
        cum_blocks.append(cj - oj + run)
        run = run + cj[BLK - 1:BLK]
    rank = jnp.concatenate(cum_blocks, axis=0)
    counts = run

    pc = jnp.floor((counts + (BLK - 1)) * (1.0 / BLK)) * BLK
    po = jnp.zeros((1, E), jnp.float32)
    acc = pc
    for sh in (1, 2, 4, 8):
        if sh < E:
            po = po
            shifted = jnp.concatenate(
                [jnp.zeros((1, sh), jnp.float32), acc[:, :E - sh]], axis=-1)
            acc = acc + shifted
    po = acc - pc

    dest = jnp.sum(onehot * (po + rank), axis=-1, keepdims=True)
    dest_ref[...] = dest.astype(jnp.int32)

    cum_end = po + pc
    bids = lax.broadcasted_iota(jnp.int32, (nb_out, 1), 0).astype(
        jnp.float32) * BLK
    bte = jnp.sum((cum_end <= bids).astype(jnp.float32), axis=-1, keepdims=True)
    bte_ref[...] = jnp.minimum(bte, E - 1).astype(jnp.int32)


def _ffn_body(bte_ref, xs_ref, w1_ref, w2_ref, out_ref):
    h = jnp.dot(xs_ref[...], w1_ref[0], preferred_element_type=jnp.float32)
    h = jnp.maximum(h, 0.0).astype(jnp.bfloat16)
    out_ref[...] = jnp.dot(
        h, w2_ref[0], preferred_element_type=jnp.float32).astype(jnp.bfloat16)


def _combine_body(g_ref, c0_ref, c1_ref, y_ref):
    g = g_ref[...]
    y_ref[...] = (g[:, 0:1] * c0_ref[...].astype(jnp.float32)
                  + g[:, 1:2] * c1_ref[...].astype(jnp.float32))


def kernel(x, Wg1, Wg2, W1, W2):
    B, S, D = x.shape
    T = B * S
    E, _, H = W1.shape
    A = 2 * T
    NB = (A + E * (BLK - 1) + BLK - 1) // BLK
    P = NB * BLK

    xf = x.reshape(T, D)

    gates, dest, bte, loss = pl.pallas_call(
        _gating_body,
        out_shape=[
            jax.ShapeDtypeStruct((T, 2), jnp.float32),
            jax.ShapeDtypeStruct((A, 1), jnp.int32),
            jax.ShapeDtypeStruct((NB, 1), jnp.int32),
            jax.ShapeDtypeStruct((1, 1), jnp.float32),
        ],
    )(xf, Wg1, Wg2)

    rr = dest.reshape(2, T)
    r0, r1 = rr[0], rr[1]
    bte_flat = bte.reshape(NB)

    info = plsc.get_sparse_core_info()
    NC, NS = info.num_cores, info.num_subcores
    NW = NC * NS
    tpw = T // NW
    mesh = plsc.VectorSubcoreMesh(core_axis_name="c", subcore_axis_name="s")

    DW = D // 2

    @functools.partial(
        pl.kernel,
        out_type=jax.ShapeDtypeStruct((P, DW), jnp.int32),
        mesh=mesh,
        scratch_types=[
            pltpu.VMEM((tpw,), jnp.int32),
            pltpu.VMEM((tpw, DW), jnp.int32),
            pltpu.SemaphoreType.DMA,
        ],
    )
    def _scatter_sc(xf_hbm, r0_hbm, r1_hbm, xs_hbm, idx_v, rows_v, sem):
        wid = lax.axis_index("s") * NC + lax.axis_index("c")
        base = wid * tpw
        pltpu.sync_copy(xf_hbm.at[pl.ds(base, tpw)], rows_v)
        pltpu.sync_copy(r0_hbm.at[pl.ds(base, tpw)], idx_v)
        pltpu.async_copy(rows_v, xs_hbm.at[idx_v], sem).wait()
        pltpu.sync_copy(r1_hbm.at[pl.ds(base, tpw)], idx_v)
        pltpu.async_copy(rows_v, xs_hbm.at[idx_v], sem).wait()

    xf_i = lax.bitcast_convert_type(
        xf.astype(jnp.bfloat16).reshape(T, DW, 2), jnp.int32)
    xs_i = _scatter_sc(xf_i, r0, r1)
    xs = lax.bitcast_convert_type(xs_i, jnp.bfloat16).reshape(P, D)

    w1b = W1.astype(jnp.bfloat16)
    w2b = W2.astype(jnp.bfloat16)
    out2 = pl.pallas_call(
        _ffn_body,
        grid_spec=pltpu.PrefetchScalarGridSpec(
            num_scalar_prefetch=1,
            grid=(NB,),
            in_specs=[
                pl.BlockSpec((BLK, D), lambda i, bte_s: (i, 0)),
                pl.BlockSpec((1, D, H), lambda i, bte_s: (bte_s[i], 0, 0)),
                pl.BlockSpec((1, H, D), lambda i, bte_s: (bte_s[i], 0, 0)),
            ],
            out_specs=pl.BlockSpec((BLK, D), lambda i, bte_s: (i, 0)),
        ),
        out_shape=jax.ShapeDtypeStruct((P, D), jnp.bfloat16),
    )(bte_flat, xs, w1b, w2b)

    @functools.partial(
        pl.kernel,
        out_type=[
            jax.ShapeDtypeStruct((T, DW), jnp.int32),
            jax.ShapeDtypeStruct((T, DW), jnp.int32),
        ],
        mesh=mesh,
        scratch_types=[
            pltpu.VMEM((tpw,), jnp.int32),
            pltpu.VMEM((tpw, DW), jnp.int32),
            pltpu.SemaphoreType.DMA,
        ],
    )
    def _gather_sc(out2_hbm, r0_hbm, r1_hbm, c0_hbm, c1_hbm, idx_v, rows_v,
                   sem):
        wid = lax.axis_index("s") * NC + lax.axis_index("c")
        base = wid * tpw
        pltpu.sync_copy(r0_hbm.at[pl.ds(base, tpw)], idx_v)
        pltpu.async_copy(out2_hbm.at[idx_v], rows_v, sem).wait()
        pltpu.sync_copy(rows_v, c0_hbm.at[pl.ds(base, tpw)])
        pltpu.sync_copy(r1_hbm.at[pl.ds(base, tpw)], idx_v)
        pltpu.async_copy(out2_hbm.at[idx_v], rows_v, sem).wait()
        pltpu.sync_copy(rows_v, c1_hbm.at[pl.ds(base, tpw)])

    out2_i = lax.bitcast_convert_type(out2.reshape(P, DW, 2), jnp.int32)
    c0_i, c1_i = _gather_sc(out2_i, r0, r1)
    c0 = lax.bitcast_convert_type(c0_i, jnp.bfloat16).reshape(T, D)
    c1 = lax.bitcast_convert_type(c1_i, jnp.bfloat16).reshape(T, D)

    TB = 128
    y = pl.pallas_call(
        _combine_body,
        grid=(T // TB,),
        in_specs=[
            pl.BlockSpec((TB, 2), lambda i: (i, 0)),
            pl.BlockSpec((TB, D), lambda i: (i, 0)),
            pl.BlockSpec((TB, D), lambda i: (i, 0)),
        ],
        out_specs=pl.BlockSpec((TB, D), lambda i: (i, 0)),
        out_shape=jax.ShapeDtypeStruct((T, D), jnp.float32),
    )(gates, c0, c1)

    return y.reshape(B, S, D), loss[0, 0]


# speedup vs baseline: 3.9987x; 3.9987x over previous
import functools

import jax
import jax.numpy as jnp
from jax import lax
from jax.experimental import pallas as pl
from jax.experimental.pallas import tpu as pltpu
from jax.experimental.pallas import tpu_sc as plsc

BLK = 128
NUM_EXPERTS = 16
NEG = -1e30


def _gating_body(xf_ref, wg1_ref, wg2_ref, gates_ref, dest_ref, bte_ref,
                 used_ref, loss_ref):
    T = xf_ref.shape[0]
    E = NUM_EXPERTS
    A = 2 * T
    nblk = A // BLK
    nb_out = bte_ref.shape[0]

    xf = xf_ref[...]
    h1 = jnp.maximum(jnp.dot(xf, wg1_ref[...]), 0.0)
    logits = jnp.dot(h1, wg2_ref[...])

    m = jnp.max(logits, axis=-1, keepdims=True)
    el = jnp.exp(logits - m)
    probs = el / jnp.sum(el, axis=-1, keepdims=True)

    iota = lax.broadcasted_iota(jnp.int32, (T, E), 1)
    i1 = jnp.min(jnp.where(logits == m, iota, E), axis=-1, keepdims=True)
    masked = jnp.where(iota == i1, NEG, logits)
    m2 = jnp.max(masked, axis=-1, keepdims=True)
    i2 = jnp.min(jnp.where(masked == m2, iota, E), axis=-1, keepdims=True)
    p1 = jnp.sum(jnp.where(iota == i1, probs, 0.0), axis=-1, keepdims=True)
    p2 = jnp.sum(jnp.where(iota == i2, probs, 0.0), axis=-1, keepdims=True)
    den = p1 + p2 + 1e-9
    gates_ref[...] = jnp.concatenate([p1 / den, p2 / den], axis=-1)

    eps = 1e-9
    p_bar = jnp.sum(probs, axis=0, keepdims=True) * (1.0 / T)
    h_marg = -jnp.sum(p_bar * jnp.log(p_bar + eps))
    h_cond = jnp.sum(-(probs * jnp.log(probs + eps))) * (1.0 / T)
    loss_ref[...] = jnp.reshape(-h_marg + h_cond, (1, 1))

    iota_a = lax.broadcasted_iota(jnp.int32, (A, E), 1)
    e_flat = jnp.concatenate([i1, i2], axis=0)
    onehot = (e_flat == iota_a).astype(jnp.float32)

    ri = lax.broadcasted_iota(jnp.int32, (BLK, BLK), 0)
    ci = lax.broadcasted_iota(jnp.int32, (BLK, BLK), 1)
    tril = (ri >= ci).astype(jnp.bfloat16)

    onehot_b = onehot.astype(jnp.bfloat16)
    cum_blocks = []
    run = jnp.zeros((1, E), jnp.float32)
    for j in range(nblk):
        oj = onehot[j * BLK:(j + 1) * BLK]
        cj = jnp.dot(tril, onehot_b[j * BLK:(j + 1) * BLK],
                     preferred_element_type=jnp.float32)
        cum_blocks.append(cj - oj + run)
        run = run + cj[BLK - 1:BLK]
    rank = jnp.concatenate(cum_blocks, axis=0)
    counts = run

    pc = jnp.floor((counts + (BLK - 1)) * (1.0 / BLK)) * BLK
    po = jnp.zeros((1, E), jnp.float32)
    acc = pc
    for sh in (1, 2, 4, 8):
        if sh < E:
            po = po
            shifted = jnp.concatenate(
                [jnp.zeros((1, sh), jnp.float32), acc[:, :E - sh]], axis=-1)
            acc = acc + shifted
    po = acc - pc

    dest = jnp.sum(onehot * (po + rank), axis=-1, keepdims=True)
    dest_ref[...] = dest.astype(jnp.int32)

    cum_end = po + pc
    bids = lax.broadcasted_iota(jnp.int32, (nb_out, 1), 0).astype(
        jnp.float32) * BLK
    bte = jnp.sum((cum_end <= bids).astype(jnp.float32), axis=-1, keepdims=True)
    bte_ref[...] = jnp.minimum(bte, E - 1).astype(jnp.int32)
    total_rows = jnp.sum(pc)
    used_ref[...] = (bids < total_rows).astype(jnp.int32)


def _ffn_body(bte_ref, used_ref, xs_ref, w1_ref, w2_ref, out_ref):
    i = pl.program_id(0)

    @pl.when(used_ref[i] != 0)
    def _():
        xb = xs_ref[...].astype(jnp.bfloat16)
        w1 = w1_ref[0].astype(jnp.bfloat16)
        h = jnp.dot(xb, w1, preferred_element_type=jnp.float32)
        h = jnp.maximum(h, 0.0).astype(jnp.bfloat16)
        w2 = w2_ref[0].astype(jnp.bfloat16)
        out_ref[...] = jnp.dot(h, w2, preferred_element_type=jnp.float32)


def kernel(x, Wg1, Wg2, W1, W2):
    B, S, D = x.shape
    T = B * S
    E, _, H = W1.shape
    A = 2 * T
    NB = (A + E * (BLK - 1) + BLK - 1) // BLK
    P = NB * BLK

    xf = x.reshape(T, D)

    gates, dest, bte, used, loss = pl.pallas_call(
        _gating_body,
        out_shape=[
            jax.ShapeDtypeStruct((T, 2), jnp.float32),
            jax.ShapeDtypeStruct((A, 1), jnp.int32),
            jax.ShapeDtypeStruct((NB, 1), jnp.int32),
            jax.ShapeDtypeStruct((NB, 1), jnp.int32),
            jax.ShapeDtypeStruct((1, 1), jnp.float32),
        ],
    )(xf, Wg1, Wg2)

    rr = dest.reshape(2, T)
    r0, r1 = rr[0], rr[1]
    bte_flat = bte.reshape(NB)
    used_flat = used.reshape(NB)

    info = plsc.get_sparse_core_info()
    NC, NS = info.num_cores, info.num_subcores
    NW = NC * NS
    tpw = T // NW
    mesh = plsc.VectorSubcoreMesh(core_axis_name="c", subcore_axis_name="s")

    @functools.partial(
        pl.kernel,
        out_type=jax.ShapeDtypeStruct((P, D), jnp.float32),
        mesh=mesh,
        scratch_types=[
            pltpu.VMEM((tpw,), jnp.int32),
            pltpu.VMEM((tpw,), jnp.int32),
            pltpu.VMEM((tpw, D), jnp.float32),
            pltpu.SemaphoreType.DMA,
        ],
    )
    def _scatter_sc(xf_hbm, r0_hbm, r1_hbm, xs_hbm, idx0_v, idx1_v, rows_v,
                    sem):
        wid = lax.axis_index("s") * NC + lax.axis_index("c")
        base = wid * tpw
        pltpu.sync_copy(xf_hbm.at[pl.ds(base, tpw)], rows_v)
        pltpu.sync_copy(r0_hbm.at[pl.ds(base, tpw)], idx0_v)
        pltpu.sync_copy(r1_hbm.at[pl.ds(base, tpw)], idx1_v)
        d0 = pltpu.async_copy(rows_v, xs_hbm.at[idx0_v], sem)
        d1 = pltpu.async_copy(rows_v, xs_hbm.at[idx1_v], sem)
        d0.wait()
        d1.wait()

    xs = _scatter_sc(xf, r0, r1)

    out2 = pl.pallas_call(
        _ffn_body,
        grid_spec=pltpu.PrefetchScalarGridSpec(
            num_scalar_prefetch=2,
            grid=(NB,),
            in_specs=[
                pl.BlockSpec((BLK, D), lambda i, bte_s, used_s: (i, 0)),
                pl.BlockSpec((1, D, H),
                             lambda i, bte_s, used_s: (bte_s[i], 0, 0)),
                pl.BlockSpec((1, H, D),
                             lambda i, bte_s, used_s: (bte_s[i], 0, 0)),
            ],
            out_specs=pl.BlockSpec((BLK, D), lambda i, bte_s, used_s: (i, 0)),
        ),
        out_shape=jax.ShapeDtypeStruct((P, D), jnp.float32),
    )(bte_flat, used_flat, xs, W1, W2)

    L = info.num_lanes
    nch = D // L
    g0 = gates[:, 0]
    g1 = gates[:, 1]

    @functools.partial(
        pl.kernel,
        out_type=jax.ShapeDtypeStruct((T, D), jnp.float32),
        mesh=mesh,
        compiler_params=pltpu.CompilerParams(needs_layout_passes=False),
        scratch_types=[
            pltpu.VMEM((tpw,), jnp.int32),
            pltpu.VMEM((tpw,), jnp.int32),
            pltpu.VMEM((tpw,), jnp.float32),
            pltpu.VMEM((tpw,), jnp.float32),
            pltpu.VMEM((tpw, D), jnp.float32),
            pltpu.VMEM((tpw, D), jnp.float32),
            pltpu.SemaphoreType.DMA,
        ],
    )
    def _combine_sc(out2_hbm, r0_hbm, r1_hbm, g0_hbm, g1_hbm, y_hbm, idx0_v,
                    idx1_v, g0_v, g1_v, rows0_v, rows1_v, sem):
        wid = lax.axis_index("s") * NC + lax.axis_index("c")
        base = wid * tpw
        pltpu.sync_copy(r0_hbm.at[pl.ds(base, tpw)], idx0_v)
        pltpu.sync_copy(r1_hbm.at[pl.ds(base, tpw)], idx1_v)
        pltpu.sync_copy(g0_hbm.at[pl.ds(base, tpw)], g0_v)
        pltpu.sync_copy(g1_hbm.at[pl.ds(base, tpw)], g1_v)
        d0 = pltpu.async_copy(out2_hbm.at[idx0_v], rows0_v, sem)
        d1 = pltpu.async_copy(out2_hbm.at[idx1_v], rows1_v, sem)
        d0.wait()
        d1.wait()

        def body_j(j, carry):
            jj = jnp.full((L,), j, jnp.int32)
            gs0 = plsc.load_gather(g0_v, [jj])
            gs1 = plsc.load_gather(g1_v, [jj])
            for c in range(nch):
                sl = pl.ds(c * L, L)
                rows0_v[j, sl] = rows0_v[j, sl] * gs0 + rows1_v[j, sl] * gs1
            return carry

        lax.fori_loop(0, tpw, body_j, 0)
        pltpu.sync_copy(rows0_v, y_hbm.at[pl.ds(base, tpw)])

    y = _combine_sc(out2, r0, r1, g0, g1)

    return y.reshape(B, S, D), loss[0, 0]
